# baseline (device time: 45899 ns/iter reference)
import functools

import jax
import jax.numpy as jnp
from jax import lax
from jax.experimental import pallas as pl
from jax.experimental.pallas import tpu as pltpu

N_DEV = 4
B = 2
SQ_LOC = 128
D_MODEL = 512
HQ = 16
HQ_LOC = 4
DH = 64
SKV = 128
BLK = 64


def kernel(x, Wq, K_ext, V_ext, Wo):
    bf16 = jnp.bfloat16
    f32 = jnp.float32

    def body(x_ref, wq_ref, k_ref, v_ref, wo_ref, out_ref,
             wq_slots, wo_slots, wq_send, wq_recv, wo_send, wo_recv):
        my = lax.axis_index("i")
        left = lax.rem(my + N_DEV - 1, N_DEV)
        right = lax.rem(my + 1, N_DEV)

        barrier_sem = pltpu.get_barrier_semaphore()
        for nbr in (left, right):
            pl.semaphore_signal(
                barrier_sem, inc=1,
                device_id=(nbr,), device_id_type=pl.DeviceIdType.MESH,
            )
        pl.semaphore_wait(barrier_sem, 2)

        wq_slots[my] = wq_ref[...].astype(bf16)
        wo_slots[my] = wo_ref[...].astype(bf16)

        for h in range(N_DEV - 1):
            s = lax.rem(my + N_DEV - h, N_DEV)
            r = lax.rem(my + N_DEV - h - 1, N_DEV)
            wq_tx = pltpu.make_async_remote_copy(
                src_ref=wq_slots.at[s], dst_ref=wq_slots.at[s],
                send_sem=wq_send.at[s], recv_sem=wq_recv.at[s],
                device_id=(right,), device_id_type=pl.DeviceIdType.MESH,
            )
            wo_tx = pltpu.make_async_remote_copy(
                src_ref=wo_slots.at[s], dst_ref=wo_slots.at[s],
                send_sem=wo_send.at[s], recv_sem=wo_recv.at[s],
                device_id=(right,), device_id_type=pl.DeviceIdType.MESH,
            )
            wq_tx.start()
            wo_tx.start()
            wq_rx = pltpu.make_async_remote_copy(
                src_ref=wq_slots.at[r], dst_ref=wq_slots.at[r],
                send_sem=wq_send.at[r], recv_sem=wq_recv.at[r],
                device_id=(right,), device_id_type=pl.DeviceIdType.MESH,
            )
            wo_rx = pltpu.make_async_remote_copy(
                src_ref=wo_slots.at[r], dst_ref=wo_slots.at[r],
                send_sem=wo_send.at[r], recv_sem=wo_recv.at[r],
                device_id=(right,), device_id_type=pl.DeviceIdType.MESH,
            )
            wq_tx.wait_send()
            wo_tx.wait_send()
            wq_rx.wait_recv()
            wo_rx.wait_recv()

        qi = lax.broadcasted_iota(jnp.int32, (SQ_LOC, SKV), 0)
        kj = lax.broadcasted_iota(jnp.int32, (SQ_LOC, SKV), 1)
        qb = my * (SQ_LOC // BLK) + qi // BLK
        kb = kj // BLK
        mask = (qb == kb) | (kb == 0) | (lax.rem(qb + kb, 3) == 0)

        for b in range(B):
            x_b = x_ref[b].astype(bf16)
            acc = jnp.zeros((SQ_LOC, D_MODEL), dtype=f32)
            for g in range(N_DEV):
                q_g = jax.lax.dot_general(
                    x_b, wq_slots[g],
                    (((1,), (0,)), ((), ())),
                    preferred_element_type=f32,
                ).astype(bf16)
                for hh in range(HQ_LOC):
                    head = g * HQ_LOC + hh
                    q = q_g[:, hh * DH:(hh + 1) * DH]
                    k = k_ref[b, :, head, :].astype(bf16)
                    v = v_ref[b, :, head, :].astype(bf16)
                    s = jax.lax.dot_general(
                        q, k, (((1,), (1,)), ((), ())),
                        preferred_element_type=f32,
                    ) * 0.125
                    s = jnp.where(mask, s, -1e9)
                    m = jnp.max(s, axis=-1, keepdims=True)
                    w = jnp.exp(s - m)
                    w = w / jnp.sum(w, axis=-1, keepdims=True)
                    ctx = jax.lax.dot_general(
                        w.astype(bf16), v, (((1,), (0,)), ((), ())),
                        preferred_element_type=f32,
                    ).astype(bf16)
                    acc = acc + jax.lax.dot_general(
                        ctx, wo_slots[g, hh * DH:(hh + 1) * DH, :],
                        (((1,), (0,)), ((), ())),
                        preferred_element_type=f32,
                    )
            out_ref[b] = acc

    return pl.pallas_call(
        body,
        out_shape=jax.ShapeDtypeStruct((B, SQ_LOC, D_MODEL), jnp.float32),
        in_specs=[pl.BlockSpec(memory_space=pltpu.VMEM)] * 5,
        out_specs=pl.BlockSpec(memory_space=pltpu.VMEM),
        scratch_shapes=[
            pltpu.VMEM((N_DEV, D_MODEL, HQ_LOC * DH), bf16),
            pltpu.VMEM((N_DEV, HQ_LOC * DH, D_MODEL), bf16),
            pltpu.SemaphoreType.DMA((N_DEV,)),
            pltpu.SemaphoreType.DMA((N_DEV,)),
            pltpu.SemaphoreType.DMA((N_DEV,)),
            pltpu.SemaphoreType.DMA((N_DEV,)),
        ],
        compiler_params=pltpu.CompilerParams(collective_id=0),
    )(x, Wq, K_ext, V_ext, Wo)


# device time: 23620 ns/iter; 1.9432x vs baseline; 1.9432x over previous
import functools
import os

import jax
import jax.numpy as jnp
from jax import lax
from jax.experimental import pallas as pl
from jax.experimental.pallas import tpu as pltpu

N_DEV = 4
B = 2
SQ_LOC = 128
D_MODEL = 512
HQ = 16
HQ_LOC = 4
DH = 64
SKV = 128
BLK = 64


def kernel(x, Wq, K_ext, V_ext, Wo):
    bf16 = jnp.bfloat16
    f32 = jnp.float32

    def body(x_ref, wq_ref, k_ref, v_ref, wo_ref, out_ref,
             wq_slots, wo_slots, wq_send, wq_recv, wo_send, wo_recv):
        my = lax.axis_index("i")
        left = lax.rem(my + N_DEV - 1, N_DEV)
        right = lax.rem(my + 1, N_DEV)

        barrier_sem = pltpu.get_barrier_semaphore()
        for nbr in (left, right):
            pl.semaphore_signal(
                barrier_sem, inc=1,
                device_id=(nbr,), device_id_type=pl.DeviceIdType.MESH,
            )
        pl.semaphore_wait(barrier_sem, 2)

        wq_slots[my] = wq_ref[...].astype(bf16)
        wo_slots[my] = wo_ref[...].astype(bf16)

        n_hops = 0 if os.environ.get("SKIP_COMM") else N_DEV - 1
        if n_hops == 0:
            for g in range(N_DEV):
                wq_slots[g] = wq_ref[...].astype(bf16)
                wo_slots[g] = wo_ref[...].astype(bf16)
        for h in range(n_hops):
            s = lax.rem(my + N_DEV - h, N_DEV)
            r = lax.rem(my + N_DEV - h - 1, N_DEV)
            wq_tx = pltpu.make_async_remote_copy(
                src_ref=wq_slots.at[s], dst_ref=wq_slots.at[s],
                send_sem=wq_send.at[s], recv_sem=wq_recv.at[s],
                device_id=(right,), device_id_type=pl.DeviceIdType.MESH,
            )
            wo_tx = pltpu.make_async_remote_copy(
                src_ref=wo_slots.at[s], dst_ref=wo_slots.at[s],
                send_sem=wo_send.at[s], recv_sem=wo_recv.at[s],
                device_id=(right,), device_id_type=pl.DeviceIdType.MESH,
            )
            wq_tx.start()
            wo_tx.start()
            wq_rx = pltpu.make_async_remote_copy(
                src_ref=wq_slots.at[r], dst_ref=wq_slots.at[r],
                send_sem=wq_send.at[r], recv_sem=wq_recv.at[r],
                device_id=(right,), device_id_type=pl.DeviceIdType.MESH,
            )
            wo_rx = pltpu.make_async_remote_copy(
                src_ref=wo_slots.at[r], dst_ref=wo_slots.at[r],
                send_sem=wo_send.at[r], recv_sem=wo_recv.at[r],
                device_id=(right,), device_id_type=pl.DeviceIdType.MESH,
            )
            wq_tx.wait_send()
            wo_tx.wait_send()
            wq_rx.wait_recv()
            wo_rx.wait_recv()

        qi = lax.broadcasted_iota(jnp.int32, (SQ_LOC, SKV), 0)
        kj = lax.broadcasted_iota(jnp.int32, (SQ_LOC, SKV), 1)
        qb = my * (SQ_LOC // BLK) + qi // BLK
        kb = kj // BLK
        mask = (qb == kb) | (kb == 0) | (lax.rem(qb + kb, 3) == 0)

        for b in range(B):
            x_b = x_ref[b].astype(bf16)
            acc = jnp.zeros((SQ_LOC, D_MODEL), dtype=f32)
            for g in range(N_DEV):
                q_g = jax.lax.dot_general(
                    x_b, wq_slots[g],
                    (((1,), (0,)), ((), ())),
                    preferred_element_type=f32,
                ).astype(bf16)
                for hh in range(HQ_LOC):
                    head = g * HQ_LOC + hh
                    q = q_g[:, hh * DH:(hh + 1) * DH]
                    k = k_ref[b, :, head, :].astype(bf16)
                    v = v_ref[b, :, head, :].astype(bf16)
                    s = jax.lax.dot_general(
                        q, k, (((1,), (1,)), ((), ())),
                        preferred_element_type=f32,
                    ) * 0.125
                    s = jnp.where(mask, s, -1e9)
                    m = jnp.max(s, axis=-1, keepdims=True)
                    w = jnp.exp(s - m)
                    w = w / jnp.sum(w, axis=-1, keepdims=True)
                    ctx = jax.lax.dot_general(
                        w.astype(bf16), v, (((1,), (0,)), ((), ())),
                        preferred_element_type=f32,
                    ).astype(bf16)
                    acc = acc + jax.lax.dot_general(
                        ctx, wo_slots[g, hh * DH:(hh + 1) * DH, :],
                        (((1,), (0,)), ((), ())),
                        preferred_element_type=f32,
                    )
            out_ref[b] = acc

    return pl.pallas_call(
        body,
        out_shape=jax.ShapeDtypeStruct((B, SQ_LOC, D_MODEL), jnp.float32),
        in_specs=[pl.BlockSpec(memory_space=pltpu.VMEM)] * 5,
        out_specs=pl.BlockSpec(memory_space=pltpu.VMEM),
        scratch_shapes=[
            pltpu.VMEM((N_DEV, D_MODEL, HQ_LOC * DH), bf16),
            pltpu.VMEM((N_DEV, HQ_LOC * DH, D_MODEL), bf16),
            pltpu.SemaphoreType.DMA((N_DEV,)),
            pltpu.SemaphoreType.DMA((N_DEV,)),
            pltpu.SemaphoreType.DMA((N_DEV,)),
            pltpu.SemaphoreType.DMA((N_DEV,)),
        ],
        compiler_params=pltpu.CompilerParams(collective_id=0),
    )(x, Wq, K_ext, V_ext, Wo)


# device time: 14431 ns/iter; 3.1806x vs baseline; 1.6368x over previous
import os

import jax
import jax.numpy as jnp
from jax import lax
from jax.experimental import pallas as pl
from jax.experimental.pallas import tpu as pltpu

N_DEV = 4
B = 2
SQ_LOC = 128
D_MODEL = 512
HQ = 16
HQ_LOC = 4
DH = 64
SKV = 128
BLK = 64
GDIM = HQ_LOC * DH


def kernel(x, Wq, K_ext, V_ext, Wo):
    bf16 = jnp.bfloat16
    f32 = jnp.float32

    x2 = x.reshape(B * SQ_LOC, D_MODEL).astype(bf16)
    kt = jnp.transpose(K_ext, (2, 0, 1, 3)).astype(bf16)
    vt = jnp.transpose(V_ext, (2, 0, 1, 3)).astype(bf16)
    wq = Wq.astype(bf16)
    wo = Wo.astype(bf16)

    def body(x_ref, wq_ref, kt_ref, vt_ref, wo_ref, out_ref,
             wq_slots, wo_slots, k_bd, v_bd,
             wq_send, wq_recv, wo_send, wo_recv):
        my = lax.axis_index("i")
        left = lax.rem(my + N_DEV - 1, N_DEV)
        right = lax.rem(my + 1, N_DEV)

        k_bd[...] = jnp.zeros((HQ_LOC * SKV, GDIM), bf16)
        v_bd[...] = jnp.zeros((HQ_LOC * SKV, GDIM), bf16)

        barrier_sem = pltpu.get_barrier_semaphore()
        for nbr in (left, right):
            pl.semaphore_signal(
                barrier_sem, inc=1,
                device_id=(nbr,), device_id_type=pl.DeviceIdType.MESH,
            )
        pl.semaphore_wait(barrier_sem, 2)

        qi = lax.broadcasted_iota(jnp.int32, (SQ_LOC, HQ_LOC * SKV), 0)
        kj = lax.broadcasted_iota(jnp.int32, (SQ_LOC, HQ_LOC * SKV), 1)
        qb = my * (SQ_LOC // BLK) + qi // BLK
        kb = lax.rem(kj, SKV) // BLK
        mask = (qb == kb) | (kb == 0) | (lax.rem(qb + kb, 3) == 0)

        def compute_group(g, wq_g, wo_g, acc):
            q_g = jax.lax.dot_general(
                x_ref[...], wq_g, (((1,), (0,)), ((), ())),
                preferred_element_type=f32,
            ).astype(bf16)
            ctxs = []
            for b in range(B):
                for hh in range(HQ_LOC):
                    head = g * HQ_LOC + hh
                    kv_h = kt_ref[head]
                    vv_h = vt_ref[head]
                    k_bd[hh * SKV:(hh + 1) * SKV, hh * DH:(hh + 1) * DH] = kv_h[b]
                    v_bd[hh * SKV:(hh + 1) * SKV, hh * DH:(hh + 1) * DH] = vv_h[b]
                q_b = q_g[b * SQ_LOC:(b + 1) * SQ_LOC, :]
                s = jax.lax.dot_general(
                    q_b, k_bd[...], (((1,), (1,)), ((), ())),
                    preferred_element_type=f32,
                ) * 0.125
                s = jnp.where(mask, s, -1e9)
                s3 = s.reshape(SQ_LOC, HQ_LOC, SKV)
                m = jnp.max(s3, axis=-1, keepdims=True)
                w = jnp.exp(s3 - m)
                w = w / jnp.sum(w, axis=-1, keepdims=True)
                w2 = w.reshape(SQ_LOC, HQ_LOC * SKV).astype(bf16)
                ctxs.append(jax.lax.dot_general(
                    w2, v_bd[...], (((1,), (0,)), ((), ())),
                    preferred_element_type=f32,
                ).astype(bf16))
            ctx = jnp.concatenate(ctxs, axis=0)
            return acc + jax.lax.dot_general(
                ctx, wo_g, (((1,), (0,)), ((), ())),
                preferred_element_type=f32,
            )

        acc = jnp.zeros((B * SQ_LOC, D_MODEL), dtype=f32)
        skip_comm = bool(os.environ.get("SKIP_COMM"))

        txs = []

        def start_hop(h, wq_src, wo_src):
            s = lax.rem(my + N_DEV - h, N_DEV)
            wq_tx = pltpu.make_async_remote_copy(
                src_ref=wq_src, dst_ref=wq_slots.at[s],
                send_sem=wq_send.at[s], recv_sem=wq_recv.at[s],
                device_id=(right,), device_id_type=pl.DeviceIdType.MESH,
            )
            wo_tx = pltpu.make_async_remote_copy(
                src_ref=wo_src, dst_ref=wo_slots.at[s],
                send_sem=wo_send.at[s], recv_sem=wo_recv.at[s],
                device_id=(right,), device_id_type=pl.DeviceIdType.MESH,
            )
            wq_tx.start()
            wo_tx.start()
            txs.extend((wq_tx, wo_tx))

        def wait_hop(h):
            r = lax.rem(my + N_DEV - h - 1, N_DEV)
            for slots, sem in ((wq_slots, wq_recv), (wo_slots, wo_recv)):
                rx = pltpu.make_async_remote_copy(
                    src_ref=slots.at[r], dst_ref=slots.at[r],
                    send_sem=sem.at[r], recv_sem=sem.at[r],
                    device_id=(right,), device_id_type=pl.DeviceIdType.MESH,
                )
                rx.wait_recv()
            return r

        if skip_comm:
            for g in range(N_DEV):
                acc = compute_group(g, wq_ref[...], wo_ref[...], acc)
        else:
            start_hop(0, wq_ref, wo_ref)
            acc = compute_group(my, wq_ref[...], wo_ref[...], acc)
            for h in range(1, N_DEV):
                r = wait_hop(h - 1)
                if h < N_DEV - 1:
                    start_hop(h, wq_slots.at[r], wo_slots.at[r])
                acc = compute_group(r, wq_slots[r], wo_slots[r], acc)
            for tx in txs:
                tx.wait_send()

        out_ref[...] = acc

    out = pl.pallas_call(
        body,
        out_shape=jax.ShapeDtypeStruct((B * SQ_LOC, D_MODEL), jnp.float32),
        in_specs=[pl.BlockSpec(memory_space=pltpu.VMEM)] * 5,
        out_specs=pl.BlockSpec(memory_space=pltpu.VMEM),
        scratch_shapes=[
            pltpu.VMEM((N_DEV, D_MODEL, GDIM), bf16),
            pltpu.VMEM((N_DEV, GDIM, D_MODEL), bf16),
            pltpu.VMEM((HQ_LOC * SKV, GDIM), bf16),
            pltpu.VMEM((HQ_LOC * SKV, GDIM), bf16),
            pltpu.SemaphoreType.DMA((N_DEV,)),
            pltpu.SemaphoreType.DMA((N_DEV,)),
            pltpu.SemaphoreType.DMA((N_DEV,)),
            pltpu.SemaphoreType.DMA((N_DEV,)),
        ],
        compiler_params=pltpu.CompilerParams(collective_id=0),
    )(x2, wq, kt, vt, wo)
    return out.reshape(B, SQ_LOC, D_MODEL)
